# Initial kernel scaffold; baseline (speedup 1.0000x reference)
import jax
import jax.numpy as jnp
from jax.experimental import pallas as pl


def kernel(observation, E, W_out):
    raise NotImplementedError("write your pallas kernel here")


# speedup vs baseline: 1.0375x; 1.0375x over previous
import functools

import jax
import jax.numpy as jnp
import numpy as np
from jax.experimental import pallas as pl
from jax.experimental.pallas import tpu as pltpu

_D = 128
_V = 100000
_L = 8
_M = 64
_VC = 16384
_NV = 7
_BIG = 2**30


def _ar_kernel(rep_ref, w_ref, g_ref, e_hbm, out_ref,
               emb_ref, tokv_ref, toks_ref, rmax_ref, ridx_ref,
               sem_gather, sem_tok):
    t = pl.program_id(0)
    v = pl.program_id(1)

    @pl.when(jnp.logical_and(t == 0, v == 0))
    def _init_emb():
        cp = pltpu.make_async_copy(e_hbm.at[pl.ds(0, 1), :],
                                   emb_ref.at[pl.ds(0, 1), :], sem_gather)
        cp.start()
        cp.wait()
        emb_ref[...] = jnp.broadcast_to(emb_ref[pl.ds(0, 1), :], (_M, _D))

    h = jnp.tanh(rep_ref[...] + emb_ref[...])
    x = jnp.dot(h, w_ref[...], preferred_element_type=jnp.float32)
    x = x + g_ref[0]
    col = jax.lax.broadcasted_iota(jnp.int32, (_M, _VC), 1)
    x = jnp.where(col + v * _VC < _V, x, -jnp.inf)
    m = jnp.max(x, axis=1, keepdims=True)
    lidx = jnp.min(jnp.where(x == m, col, _BIG), axis=1,
                   keepdims=True) + v * _VC

    @pl.when(v == 0)
    def _first_chunk():
        rmax_ref[...] = m
        ridx_ref[...] = lidx

    @pl.when(v > 0)
    def _merge_chunk():
        better = m > rmax_ref[...]
        ridx_ref[...] = jnp.where(better, lidx, ridx_ref[...])
        rmax_ref[...] = jnp.maximum(m, rmax_ref[...])

    @pl.when(v == _NV - 1)
    def _finish_step():
        tok = ridx_ref[...]
        out_ref[0, 0, :] = tok.reshape(_M)
        tokv_ref[...] = tok

        @pl.when(t < _L - 1)
        def _gather_next():
            cp = pltpu.make_async_copy(tokv_ref, toks_ref, sem_tok)
            cp.start()
            cp.wait()

            def _start(i, _):
                idx = toks_ref[i, 0]
                pltpu.make_async_copy(e_hbm.at[pl.ds(idx, 1), :],
                                      emb_ref.at[pl.ds(i, 1), :],
                                      sem_gather).start()
                return 0

            jax.lax.fori_loop(0, _M, _start, 0)

            def _wait(i, _):
                pltpu.make_async_copy(e_hbm.at[pl.ds(0, 1), :],
                                      emb_ref.at[pl.ds(i, 1), :],
                                      sem_gather).wait()
                return 0

            jax.lax.fori_loop(0, _M, _wait, 0)


@functools.partial(jax.jit, static_argnames=("interpret",))
def _run(all_inputs, E, W_out, g, interpret=False):
    toks = pl.pallas_call(
        _ar_kernel,
        grid=(_L, _NV),
        in_specs=[
            pl.BlockSpec((_M, _D), lambda t, v: (0, 0)),
            pl.BlockSpec((_D, _VC), lambda t, v: (0, v)),
            pl.BlockSpec((1, _M, _VC), lambda t, v: (t, 0, v)),
            pl.BlockSpec(memory_space=pl.MemorySpace.ANY),
        ],
        out_specs=pl.BlockSpec((1, 1, _M), lambda t, v: (t, 0, 0)),
        out_shape=jax.ShapeDtypeStruct((_L, 1, _M), jnp.int32),
        scratch_shapes=[
            pltpu.VMEM((_M, _D), jnp.float32),
            pltpu.VMEM((_M, 1), jnp.int32),
            pltpu.SMEM((_M, 1), jnp.int32),
            pltpu.VMEM((_M, 1), jnp.float32),
            pltpu.VMEM((_M, 1), jnp.int32),
            pltpu.SemaphoreType.DMA,
            pltpu.SemaphoreType.DMA,
        ],
        compiler_params=pltpu.CompilerParams(
            dimension_semantics=("arbitrary", "arbitrary"),
        ),
        interpret=interpret,
    )(all_inputs, W_out, g, E)
    return toks


def kernel(observation, E, W_out):
    batch = observation.shape[0]
    ipo = observation.shape[1] // _D
    all_inputs = observation.reshape(batch * ipo, _D)
    base_key = jax.random.key(1)
    g = jnp.stack([
        jax.random.gumbel(jax.random.fold_in(base_key, t), (_M, _V), jnp.float32)
        for t in range(_L)
    ])
    toks = _run(all_inputs, E, W_out, g)
    seqs = jnp.transpose(toks.reshape(_L, _M))
    seq_supp_batch = seqs.reshape(batch, ipo, _L)
    length_supp_batch = jnp.full((batch, ipo), _L, dtype=jnp.int32)
    return seq_supp_batch, length_supp_batch
